# Initial kernel scaffold; baseline (speedup 1.0000x reference)
import jax
import jax.numpy as jnp
from jax.experimental import pallas as pl


def kernel(x, p, x_old, p_old, W_up, gamma_up, beta_up, W_lat, gamma_lat, beta_lat):
    raise NotImplementedError("write your pallas kernel here")


# speedup vs baseline: 31.4155x; 31.4155x over previous
import functools

import jax
import jax.numpy as jnp
from jax.experimental import pallas as pl
from jax.experimental.pallas import tpu as pltpu

EPS_BN = 1e-5
MB = 512


def _stage1_body(x_ref, xold_ref, wup_ref, wlat_ref,
                 zup_ref, zlat_ref, stats_ref):
    b = pl.program_id(0)
    m = pl.program_id(1)

    @pl.when(jnp.logical_and(b == 0, m == 0))
    def _init():
        stats_ref[...] = jnp.zeros_like(stats_ref)

    @pl.when(m == 0)
    def _up():
        zup = jax.lax.dot_general(
            x_ref[0], wup_ref[...], (((1,), (1,)), ((), ())),
            preferred_element_type=jnp.float32)
        zup_ref[0] = zup
        stats_ref[0:1, :] += jnp.sum(zup, axis=0, keepdims=True)
        stats_ref[1:2, :] += jnp.sum(zup * zup, axis=0, keepdims=True)

    zlat = jax.lax.dot_general(
        xold_ref[0], wlat_ref[...], (((1,), (1,)), ((), ())),
        preferred_element_type=jnp.float32)
    zlat_ref[0] = zlat
    stats_ref[2:3, :] += jnp.sum(zlat, axis=0, keepdims=True)
    stats_ref[3:4, :] += jnp.sum(zlat * zlat, axis=0, keepdims=True)


def _stage2_body(zup_ref, zlat_ref, pt_ref, pold_ref, aff_ref,
                 y_ref, h_ref):
    m = pl.program_id(1)
    n = zup_ref.shape[1]

    @pl.when(m == 0)
    def _norm_up():
        h_ref[...] = jnp.maximum(
            zup_ref[0] * aff_ref[0:1, :] + aff_ref[1:2, :], 0.0)

    pold = pold_ref[0]
    pt = pt_ref[0]
    d0 = pold[:, 0:1] - pt[0:1, :]
    d1 = pold[:, 1:2] - pt[1:2, :]
    d2c = pold[:, 2:3] - pt[2:3, :]
    dist = d0 * d0 + d1 * d1 + d2c * d2c

    idx = jax.lax.broadcasted_iota(jnp.int32, dist.shape, 1)
    d = dist
    a = jnp.zeros_like(dist)
    wsum = jnp.zeros((dist.shape[0], 1), jnp.float32)
    for _ in range(3):
        mj = jnp.min(d, axis=1, keepdims=True)
        ij = jnp.min(jnp.where(d == mj, idx, n), axis=1, keepdims=True)
        sel = idx == ij
        wj = 1.0 / jnp.maximum(mj, 1e-16)
        a = a + jnp.where(sel, wj, 0.0)
        wsum = wsum + wj
        d = jnp.where(sel, jnp.float32(jnp.inf), d)
    a = a / wsum

    interp = jax.lax.dot_general(
        a, h_ref[...], (((1,), (0,)), ((), ())),
        preferred_element_type=jnp.float32)
    lat = jnp.maximum(zlat_ref[0] * aff_ref[2:3, :] + aff_ref[3:4, :], 0.0)
    y_ref[0] = interp + lat


@functools.partial(jax.jit, static_argnames=())
def kernel(x, p, x_old, p_old, W_up, gamma_up, beta_up,
           W_lat, gamma_lat, beta_lat):
    B, N, Cin = x.shape
    M = p_old.shape[1]
    Cout = W_up.shape[0]
    nmb = M // MB

    grid = (B, nmb)
    zup, zlat, stats = pl.pallas_call(
        _stage1_body,
        grid=grid,
        in_specs=[
            pl.BlockSpec((1, N, Cin), lambda b, m: (b, 0, 0)),
            pl.BlockSpec((1, MB, Cout), lambda b, m: (b, m, 0)),
            pl.BlockSpec((Cout, Cin), lambda b, m: (0, 0)),
            pl.BlockSpec((Cout, Cout), lambda b, m: (0, 0)),
        ],
        out_specs=[
            pl.BlockSpec((1, N, Cout), lambda b, m: (b, 0, 0)),
            pl.BlockSpec((1, MB, Cout), lambda b, m: (b, m, 0)),
            pl.BlockSpec((8, Cout), lambda b, m: (0, 0)),
        ],
        out_shape=[
            jax.ShapeDtypeStruct((B, N, Cout), jnp.float32),
            jax.ShapeDtypeStruct((B, M, Cout), jnp.float32),
            jax.ShapeDtypeStruct((8, Cout), jnp.float32),
        ],
        compiler_params=pltpu.CompilerParams(
            dimension_semantics=("arbitrary", "arbitrary")),
    )(x, x_old, W_up, W_lat)

    n_up = jnp.float32(B * N)
    n_lat = jnp.float32(B * M)
    mean_up = stats[0] / n_up
    var_up = jnp.maximum(stats[1] / n_up - mean_up * mean_up, 0.0)
    scale_up = gamma_up * jax.lax.rsqrt(var_up + EPS_BN)
    shift_up = beta_up - mean_up * scale_up
    mean_lat = stats[2] / n_lat
    var_lat = jnp.maximum(stats[3] / n_lat - mean_lat * mean_lat, 0.0)
    scale_lat = gamma_lat * jax.lax.rsqrt(var_lat + EPS_BN)
    shift_lat = beta_lat - mean_lat * scale_lat
    aff = jnp.stack([scale_up, shift_up, scale_lat, shift_lat], axis=0)
    aff = jnp.concatenate([aff, jnp.zeros((4, Cout), jnp.float32)], axis=0)

    p_t = jnp.transpose(p, (0, 2, 1))

    y = pl.pallas_call(
        _stage2_body,
        grid=grid,
        in_specs=[
            pl.BlockSpec((1, N, Cout), lambda b, m: (b, 0, 0)),
            pl.BlockSpec((1, MB, Cout), lambda b, m: (b, m, 0)),
            pl.BlockSpec((1, 3, N), lambda b, m: (b, 0, 0)),
            pl.BlockSpec((1, MB, 3), lambda b, m: (b, m, 0)),
            pl.BlockSpec((8, Cout), lambda b, m: (0, 0)),
        ],
        out_specs=pl.BlockSpec((1, MB, Cout), lambda b, m: (b, m, 0)),
        out_shape=jax.ShapeDtypeStruct((B, M, Cout), jnp.float32),
        scratch_shapes=[pltpu.VMEM((N, Cout), jnp.float32)],
        compiler_params=pltpu.CompilerParams(
            dimension_semantics=("arbitrary", "arbitrary")),
    )(zup, zlat, p_t, p_old, aff)

    return (y, p_old)
